# Initial kernel scaffold; baseline (speedup 1.0000x reference)
import jax
import jax.numpy as jnp
from jax.experimental import pallas as pl


def kernel(x, edge_index, edge_weight, W, b):
    raise NotImplementedError("write your pallas kernel here")


# speedup vs baseline: 3.7313x; 3.7313x over previous
import dataclasses
import functools

import jax
import jax.numpy as jnp
from jax import lax
from jax.experimental import pallas as pl
from jax.experimental.pallas import tpu as pltpu
from jax.experimental.pallas import tpu_sc as plsc

_N = 10000
_E = 320000
_D = 128
_C = 128
_NSUB = 16
_NTILES = 32
_CPT = 79
_EPAD = _C * _NTILES * _CPT
_NPAD = 10112
_RPT = _NPAD // _NSUB


def _sc_scatter(src2, dst2, w2, x, zslab):
    mesh = plsc.VectorSubcoreMesh(core_axis_name="c", subcore_axis_name="s")
    cp = pltpu.CompilerParams()
    if "needs_layout_passes" in pltpu.CompilerParams.__dataclass_fields__:
        cp = dataclasses.replace(cp, needs_layout_passes=False)

    @functools.partial(
        pl.kernel,
        out_type=jax.ShapeDtypeStruct((2 * _NPAD, _D), jnp.float32),
        mesh=mesh,
        compiler_params=cp,
        scratch_types=[
            pltpu.VMEM_SHARED((_NPAD, _D), jnp.float32),
            pltpu.VMEM((1, _C), jnp.int32),
            pltpu.VMEM((1, _C), jnp.int32),
            pltpu.VMEM((_C,), jnp.float32),
            pltpu.VMEM((_C, _D), jnp.float32),
            pltpu.SemaphoreType.DMA,
        ],
    )
    def k(src_hbm, dst_hbm, w_hbm, x_hbm, z_hbm, out_hbm,
          acc, sidx, didx, wbuf, rows, sem):
        c = lax.axis_index("c")
        s = lax.axis_index("s")
        wid = c * _NSUB + s
        slab = s * _RPT

        pltpu.sync_copy(z_hbm, acc.at[pl.ds(slab, _RPT)])
        plsc.subcore_barrier()

        base_chunk = wid * _CPT

        @pl.loop(0, _CPT)
        def _(ci):
            row = base_chunk + ci
            pltpu.sync_copy(src_hbm.at[row], sidx.at[0])
            pltpu.sync_copy(dst_hbm.at[row], didx.at[0])
            pltpu.sync_copy(w_hbm.at[row], wbuf)
            pltpu.async_copy(x_hbm.at[sidx.at[0]], rows, sem).wait()

            @pl.loop(0, _C)
            def _(e):
                wv = plsc.load_gather(wbuf, [jnp.full((16,), e, jnp.int32)])
                for kk in range(8):
                    sl = (e, pl.ds(kk * 16, 16))
                    rows[sl] = rows[sl] * wv

            pltpu.sync_copy(rows, acc.at[didx.at[0]], add=True)

        plsc.subcore_barrier()
        out_base = c * _NPAD + slab
        pltpu.sync_copy(acc.at[pl.ds(slab, _RPT)],
                        out_hbm.at[pl.ds(out_base, _RPT)])

    return k(src2, dst2, w2, x, zslab)


_BLK = 1000


def _tc_finish(x, partials, Wt, b2):
    def body(x_ref, p_ref, wt_ref, b_ref, o_ref):
        g = p_ref[0] + p_ref[1]
        nrm = jnp.sqrt(jnp.sum(g * g, axis=1, keepdims=True))
        gn = g / jnp.maximum(nrm, 1e-12)
        h = jnp.concatenate([x_ref[...], gn], axis=1)
        o = lax.dot_general(h, wt_ref[...], (((1,), (0,)), ((), ())),
                            preferred_element_type=jnp.float32,
                            precision=lax.Precision.HIGHEST) + b_ref[...]
        nrm2 = jnp.sqrt(jnp.sum(o * o, axis=1, keepdims=True))
        o_ref[...] = o / jnp.maximum(nrm2, 1e-12)

    return pl.pallas_call(
        body,
        grid=(_N // _BLK,),
        in_specs=[
            pl.BlockSpec((_BLK, _D), lambda i: (i, 0)),
            pl.BlockSpec((2, _BLK, _D), lambda i: (0, i, 0)),
            pl.BlockSpec((2 * _D, _D), lambda i: (0, 0)),
            pl.BlockSpec((1, _D), lambda i: (0, 0)),
        ],
        out_specs=pl.BlockSpec((_BLK, _D), lambda i: (i, 0)),
        out_shape=jax.ShapeDtypeStruct((_N, _D), jnp.float32),
    )(x, partials, Wt, b2)


def kernel(x, edge_index, edge_weight, W, b):
    src = edge_index[0]
    dst = edge_index[1]
    pad = _EPAD - _E
    src2 = jnp.pad(src, (0, pad)).reshape(_NTILES * _CPT, _C)
    dst2 = jnp.pad(dst, (0, pad)).reshape(_NTILES * _CPT, _C)
    w2 = jnp.pad(edge_weight, (0, pad)).reshape(_NTILES * _CPT, _C)
    zslab = jnp.zeros((_RPT, _D), jnp.float32)
    partials = _sc_scatter(src2, dst2, w2, x, zslab).reshape(2, _NPAD, _D)[:, :_N]
    return _tc_finish(x, partials, W.T, b[None, :])
